# Initial kernel scaffold; baseline (speedup 1.0000x reference)
import jax
import jax.numpy as jnp
from jax.experimental import pallas as pl


def kernel(features, W1, b1, W2, b2):
    raise NotImplementedError("write your pallas kernel here")


# speedup vs baseline: 4.0182x; 4.0182x over previous
import functools

import jax
import jax.numpy as jnp
from jax import lax
from jax.experimental import pallas as pl
from jax.experimental.pallas import tpu as pltpu
from jax.experimental.pallas import tpu_sc as plsc

N = 10000
D = 512
K = 33
KPAD = 64
NPAD = 10240
R = 128
NB = NPAD // R
E = N * K
NW = 32
CHUNK = 10320
EPAD = CHUNK * NW
NEG = float("-inf")


def _mlp_body(f_ref, w1t_ref, b1_ref, w2t_ref, b2_ref, emb_ref, xn_ref):
    f = f_ref[...]
    h = jnp.dot(f, w1t_ref[...], preferred_element_type=jnp.float32)
    h = jnp.maximum(h + b1_ref[...], 0.0)
    e = jnp.dot(h, w2t_ref[...], preferred_element_type=jnp.float32)
    e = e + b2_ref[...]
    emb_ref[...] = e
    nrm = jnp.sqrt(jnp.sum(e * e, axis=1, keepdims=True))
    xn_ref[...] = e / jnp.maximum(nrm, 1e-12)


def _mlp_call(f, w1t, b1r, w2t, b2r):
    return pl.pallas_call(
        _mlp_body,
        grid=(NB,),
        in_specs=[
            pl.BlockSpec((R, D), lambda i: (i, 0)),
            pl.BlockSpec((D, D), lambda i: (0, 0)),
            pl.BlockSpec((1, D), lambda i: (0, 0)),
            pl.BlockSpec((D, D), lambda i: (0, 0)),
            pl.BlockSpec((1, D), lambda i: (0, 0)),
        ],
        out_specs=[
            pl.BlockSpec((R, D), lambda i: (i, 0)),
            pl.BlockSpec((R, D), lambda i: (i, 0)),
        ],
        out_shape=[
            jax.ShapeDtypeStruct((NPAD, D), jnp.float32),
            jax.ShapeDtypeStruct((NPAD, D), jnp.float32),
        ],
    )(f, w1t, b1r, w2t, b2r)


def _topk_body(xb_ref, xnt_ref, vals_ref, inds_ref, rn_ref, sim0_ref, sim_ref):
    b = pl.program_id(0)
    sim0_ref[...] = jnp.dot(xb_ref[...], xnt_ref[...],
                            preferred_element_type=jnp.float32)
    colid = lax.broadcasted_iota(jnp.int32, (R, NPAD), 1)
    sim_ref[...] = jnp.where(colid < N, sim0_ref[...], NEG)

    def extract(j, nr):
        sim = sim_ref[...]
        m = jnp.max(sim, axis=1)
        idx = jnp.argmax(sim, axis=1).astype(jnp.int32)
        vals_ref[0, j, :] = m
        inds_ref[0, j, :] = idx
        sim_ref[...] = jnp.where(colid == idx[:, None], NEG, sim)
        return nr + m

    nr = lax.fori_loop(0, K, extract, jnp.zeros((R,), jnp.float32))

    rowid = lax.broadcasted_iota(jnp.int32, (R, NPAD), 0) + b * R
    rowok = rowid < N
    extracted = (sim_ref[...] == NEG) & (colid < N) & rowok
    contrib = jnp.where(extracted, sim0_ref[...], 0.0)
    contrib = contrib + jnp.where((colid == rowid) & rowok, nr[:, None], 0.0)
    part = jnp.sum(contrib, axis=0)

    @pl.when(b == 0)
    def _init():
        rn_ref[...] = jnp.zeros_like(rn_ref)

    rn_ref[0, :] = rn_ref[0, :] + part

    @pl.when(b == NB - 1)
    def _finish():
        rn_ref[0, :] = lax.rsqrt(rn_ref[0, :])


def _topk_call(xn, xnt):
    return pl.pallas_call(
        _topk_body,
        grid=(NB,),
        in_specs=[
            pl.BlockSpec((R, D), lambda i: (i, 0)),
            pl.BlockSpec((D, NPAD), lambda i: (0, 0)),
        ],
        out_specs=[
            pl.BlockSpec((1, KPAD, R), lambda i: (i, 0, 0)),
            pl.BlockSpec((1, KPAD, R), lambda i: (i, 0, 0)),
            pl.BlockSpec((1, NPAD), lambda i: (0, 0)),
        ],
        out_shape=[
            jax.ShapeDtypeStruct((NB, KPAD, R), jnp.float32),
            jax.ShapeDtypeStruct((NB, KPAD, R), jnp.int32),
            jax.ShapeDtypeStruct((1, NPAD), jnp.float32),
        ],
        scratch_shapes=[
            pltpu.VMEM((R, NPAD), jnp.float32),
            pltpu.VMEM((R, NPAD), jnp.float32),
        ],
    )(xn, xnt)


def _edge_body(vals_hbm, rows_hbm, cols_hbm, rn_hbm, out_hbm,
               vals_v, rows_v, cols_v, rn_v, out_v):
    wid = lax.axis_index("s") * 2 + lax.axis_index("c")
    base = wid * CHUNK
    pltpu.sync_copy(rn_hbm, rn_v)
    pltpu.sync_copy(vals_hbm.at[pl.ds(base, CHUNK)], vals_v)
    pltpu.sync_copy(rows_hbm.at[pl.ds(base, CHUNK)], rows_v)
    pltpu.sync_copy(cols_hbm.at[pl.ds(base, CHUNK)], cols_v)

    def body(i, carry):
        s = pl.ds(i * 16, 16)
        c = cols_v[s]
        r = rows_v[s]
        v = vals_v[s]
        rc = plsc.load_gather(rn_v, [c])
        rr = plsc.load_gather(rn_v, [r])
        out_v[s] = v * rc * rr
        return carry

    lax.fori_loop(0, CHUNK // 16, body, 0)
    pltpu.sync_copy(out_v, out_hbm.at[pl.ds(base, CHUNK)])


def _edge_call(*args):
    call = functools.partial(
        pl.kernel,
        mesh=plsc.VectorSubcoreMesh(core_axis_name="c", subcore_axis_name="s"),
        compiler_params=pltpu.CompilerParams(needs_layout_passes=False),
        out_type=jax.ShapeDtypeStruct((EPAD,), jnp.float32),
        scratch_types=[
            pltpu.VMEM((CHUNK,), jnp.float32),
            pltpu.VMEM((CHUNK,), jnp.int32),
            pltpu.VMEM((CHUNK,), jnp.int32),
            pltpu.VMEM((NPAD,), jnp.float32),
            pltpu.VMEM((CHUNK,), jnp.float32),
        ],
    )(_edge_body)
    return call(*args)


def kernel(features, W1, b1, W2, b2):
    f = jnp.pad(features, ((0, NPAD - N), (0, 0)))
    emb, xn = _mlp_call(f, W1.T, b1.reshape(1, D), W2.T, b2.reshape(1, D))
    vals3, inds3, rn = _topk_call(xn, xn.T)

    vals = jnp.transpose(vals3, (0, 2, 1)).reshape(NPAD, KPAD)[:N, :K]
    inds = jnp.transpose(inds3, (0, 2, 1)).reshape(NPAD, KPAD)[:N, :K]
    rows = jnp.repeat(jnp.arange(N, dtype=jnp.int32), K)
    cols = inds.reshape(-1)
    vflat = vals.reshape(-1)

    w = _edge_call(
        jnp.pad(vflat, (0, EPAD - E)),
        jnp.pad(rows, (0, EPAD - E)),
        jnp.pad(cols, (0, EPAD - E)),
        rn.reshape(NPAD),
    )
    edge_weight = w[:E]
    edge_index = jnp.stack([rows, cols])
    return (edge_index, edge_weight, emb[:N])


# speedup vs baseline: 4.7180x; 1.1741x over previous
import functools

import jax
import jax.numpy as jnp
from jax import lax
from jax.experimental import pallas as pl
from jax.experimental.pallas import tpu as pltpu
from jax.experimental.pallas import tpu_sc as plsc

N = 10000
D = 512
K = 33
KPAD = 64
NPAD = 10240
R = 128
NB = NPAD // R
E = N * K
NW = 32
CHUNK = 10320
EPAD = CHUNK * NW
NEG = float("-inf")


def _mlp_body(f_ref, w1t_ref, b1_ref, w2t_ref, b2_ref, emb_ref, xn_ref):
    f = f_ref[...]
    h = jnp.dot(f, w1t_ref[...], preferred_element_type=jnp.float32)
    h = jnp.maximum(h + b1_ref[...], 0.0)
    e = jnp.dot(h, w2t_ref[...], preferred_element_type=jnp.float32)
    e = e + b2_ref[...]
    emb_ref[...] = e
    nrm = jnp.sqrt(jnp.sum(e * e, axis=1, keepdims=True))
    xn_ref[...] = e / jnp.maximum(nrm, 1e-12)


def _mlp_call(f, w1t, b1r, w2t, b2r):
    return pl.pallas_call(
        _mlp_body,
        grid=(NB,),
        in_specs=[
            pl.BlockSpec((R, D), lambda i: (i, 0)),
            pl.BlockSpec((D, D), lambda i: (0, 0)),
            pl.BlockSpec((1, D), lambda i: (0, 0)),
            pl.BlockSpec((D, D), lambda i: (0, 0)),
            pl.BlockSpec((1, D), lambda i: (0, 0)),
        ],
        out_specs=[
            pl.BlockSpec((R, D), lambda i: (i, 0)),
            pl.BlockSpec((R, D), lambda i: (i, 0)),
        ],
        out_shape=[
            jax.ShapeDtypeStruct((NPAD, D), jnp.float32),
            jax.ShapeDtypeStruct((NPAD, D), jnp.float32),
        ],
    )(f, w1t, b1r, w2t, b2r)


def _topk_body(xb_ref, xnt_ref, vals_ref, inds_ref, rn_ref, sim0_ref, sim_ref):
    b = pl.program_id(0)
    sim0_ref[...] = jnp.dot(xb_ref[...], xnt_ref[...],
                            preferred_element_type=jnp.float32)
    colid = lax.broadcasted_iota(jnp.int32, (R, NPAD), 1)
    sim_ref[...] = jnp.where(colid < N, sim0_ref[...], NEG)

    def extract(j, nr):
        sim = sim_ref[...]
        cid = lax.broadcasted_iota(jnp.int32, (R, NPAD), 1)
        m = jnp.max(sim, axis=1)
        idx = jnp.min(jnp.where(sim == m[:, None], cid, NPAD), axis=1)
        vals_ref[0, j, :] = m
        inds_ref[0, j, :] = idx
        sim_ref[...] = jnp.where(cid == idx[:, None], NEG, sim)
        return nr + m

    nr = lax.fori_loop(0, K, extract, jnp.zeros((R,), jnp.float32))

    rowid = lax.broadcasted_iota(jnp.int32, (R, NPAD), 0) + b * R
    rowok = rowid < N
    extracted = (sim_ref[...] == NEG) & (colid < N) & rowok
    contrib = jnp.where(extracted, sim0_ref[...], 0.0)
    contrib = contrib + jnp.where((colid == rowid) & rowok, nr[:, None], 0.0)
    part = jnp.sum(contrib, axis=0)

    @pl.when(b == 0)
    def _init():
        rn_ref[...] = jnp.zeros_like(rn_ref)

    rn_ref[0, :] = rn_ref[0, :] + part

    @pl.when(b == NB - 1)
    def _finish():
        rn_ref[0, :] = lax.rsqrt(rn_ref[0, :])


def _topk_call(xn, xnt):
    return pl.pallas_call(
        _topk_body,
        grid=(NB,),
        in_specs=[
            pl.BlockSpec((R, D), lambda i: (i, 0)),
            pl.BlockSpec((D, NPAD), lambda i: (0, 0)),
        ],
        out_specs=[
            pl.BlockSpec((1, KPAD, R), lambda i: (i, 0, 0)),
            pl.BlockSpec((1, KPAD, R), lambda i: (i, 0, 0)),
            pl.BlockSpec((1, NPAD), lambda i: (0, 0)),
        ],
        out_shape=[
            jax.ShapeDtypeStruct((NB, KPAD, R), jnp.float32),
            jax.ShapeDtypeStruct((NB, KPAD, R), jnp.int32),
            jax.ShapeDtypeStruct((1, NPAD), jnp.float32),
        ],
        scratch_shapes=[
            pltpu.VMEM((R, NPAD), jnp.float32),
            pltpu.VMEM((R, NPAD), jnp.float32),
        ],
    )(xn, xnt)


def _edge_body(vals_hbm, rows_hbm, cols_hbm, rn_hbm, out_hbm,
               vals_v, rows_v, cols_v, rn_v, out_v):
    wid = lax.axis_index("s") * 2 + lax.axis_index("c")
    base = wid * CHUNK
    pltpu.sync_copy(rn_hbm, rn_v)
    pltpu.sync_copy(vals_hbm.at[pl.ds(base, CHUNK)], vals_v)
    pltpu.sync_copy(rows_hbm.at[pl.ds(base, CHUNK)], rows_v)
    pltpu.sync_copy(cols_hbm.at[pl.ds(base, CHUNK)], cols_v)

    def body(i, carry):
        s = pl.ds(i * 16, 16)
        c = cols_v[s]
        r = rows_v[s]
        v = vals_v[s]
        rc = plsc.load_gather(rn_v, [c])
        rr = plsc.load_gather(rn_v, [r])
        out_v[s] = v * rc * rr
        return carry

    lax.fori_loop(0, CHUNK // 16, body, 0)
    pltpu.sync_copy(out_v, out_hbm.at[pl.ds(base, CHUNK)])


def _edge_call(*args):
    call = functools.partial(
        pl.kernel,
        mesh=plsc.VectorSubcoreMesh(core_axis_name="c", subcore_axis_name="s"),
        compiler_params=pltpu.CompilerParams(needs_layout_passes=False),
        out_type=jax.ShapeDtypeStruct((EPAD,), jnp.float32),
        scratch_types=[
            pltpu.VMEM((CHUNK,), jnp.float32),
            pltpu.VMEM((CHUNK,), jnp.int32),
            pltpu.VMEM((CHUNK,), jnp.int32),
            pltpu.VMEM((NPAD,), jnp.float32),
            pltpu.VMEM((CHUNK,), jnp.float32),
        ],
    )(_edge_body)
    return call(*args)


def kernel(features, W1, b1, W2, b2):
    f = jnp.pad(features, ((0, NPAD - N), (0, 0)))
    emb, xn = _mlp_call(f, W1.T, b1.reshape(1, D), W2.T, b2.reshape(1, D))
    vals3, inds3, rn = _topk_call(xn, xn.T)

    vals = jnp.transpose(vals3, (0, 2, 1)).reshape(NPAD, KPAD)[:N, :K]
    inds = jnp.transpose(inds3, (0, 2, 1)).reshape(NPAD, KPAD)[:N, :K]
    rows = jnp.repeat(jnp.arange(N, dtype=jnp.int32), K)
    cols = inds.reshape(-1)
    vflat = vals.reshape(-1)

    w = _edge_call(
        jnp.pad(vflat, (0, EPAD - E)),
        jnp.pad(rows, (0, EPAD - E)),
        jnp.pad(cols, (0, EPAD - E)),
        rn.reshape(NPAD),
    )
    edge_weight = w[:E]
    edge_index = jnp.stack([rows, cols])
    return (edge_index, edge_weight, emb[:N])
